# baseline (speedup 1.0000x reference)
import jax
import jax.numpy as jnp
from jax.experimental import pallas as pl
from jax.experimental.pallas import tpu as pltpu

N = 10000
D = 128
F = 4
G = 64
H = 512
C = 10

BM = 200
NSTREAM = 2
M_BLOCKS = N // (BM * NSTREAM)


def _seg_max_local(h_blk, ids, prev):
    gids = jax.lax.broadcasted_iota(jnp.int32, (BM, G), 1)
    oh = ids == gids
    cols = [jnp.max(jnp.where(oh, h_blk[:, f:f + 1], 0.0),
                    axis=0, keepdims=True) for f in range(F)]
    return jnp.maximum(prev, jnp.concatenate(cols, axis=0))


def _fused_kernel(x_ref, wg_ref, f0_ref, f1_ref, i0_ref, i1_ref,
                  wh_ref, bh_ref, wc_ref, bc_ref, out_ref,
                  xw_ref, pooled_ref):
    m = pl.program_id(0)

    @pl.when(m == 0)
    def _init():
        xw_ref[...] = jnp.dot(x_ref[...], wg_ref[...],
                              preferred_element_type=jnp.float32)
        pooled_ref[...] = jnp.zeros_like(pooled_ref)

    xw = xw_ref[...]
    h0 = jnp.maximum(
        jnp.dot(f0_ref[...], xw, preferred_element_type=jnp.float32), 0.0)
    h1 = jnp.maximum(
        jnp.dot(f1_ref[...], xw, preferred_element_type=jnp.float32), 0.0)
    acc = _seg_max_local(h0, i0_ref[...], pooled_ref[...])
    acc = _seg_max_local(h1, i1_ref[...], acc)
    pooled_ref[...] = acc

    @pl.when(m == M_BLOCKS - 1)
    def _head():
        pooled_t = pooled_ref[...]
        z = jnp.maximum(
            jax.lax.dot_general(pooled_t, wh_ref[...],
                                (((0,), (0,)), ((), ())),
                                preferred_element_type=jnp.float32)
            + bh_ref[...], 0.0)
        logits = jnp.dot(z, wc_ref[...],
                         preferred_element_type=jnp.float32) + bc_ref[...]
        mx = jnp.max(logits, axis=-1, keepdims=True)
        e = jnp.exp(logits - mx)
        out_ref[...] = e / jnp.sum(e, axis=-1, keepdims=True)


@jax.jit
def _run(filtre, X, ids2, W_gcn, W_h, b_h, W_c, b_c):
    ns = NSTREAM
    return pl.pallas_call(
        _fused_kernel,
        grid=(M_BLOCKS,),
        in_specs=[
            pl.BlockSpec((N, D), lambda m: (0, 0)),
            pl.BlockSpec((D, F), lambda m: (0, 0)),
            pl.BlockSpec((BM, N), lambda m: (ns * m, 0)),
            pl.BlockSpec((BM, N), lambda m: (ns * m + 1, 0)),
            pl.BlockSpec((BM, 1), lambda m: (ns * m, 0)),
            pl.BlockSpec((BM, 1), lambda m: (ns * m + 1, 0)),
            pl.BlockSpec((F, H), lambda m: (0, 0)),
            pl.BlockSpec((1, H), lambda m: (0, 0)),
            pl.BlockSpec((H, C), lambda m: (0, 0)),
            pl.BlockSpec((1, C), lambda m: (0, 0)),
        ],
        out_specs=pl.BlockSpec((G, C), lambda m: (0, 0)),
        out_shape=jax.ShapeDtypeStruct((G, C), jnp.float32),
        scratch_shapes=[
            pltpu.VMEM((N, F), jnp.float32),
            pltpu.VMEM((F, G), jnp.float32),
        ],
    )(X, W_gcn, filtre, filtre, ids2, ids2, W_h, b_h, W_c, b_c)


def kernel(filtre, X, node_indicator, W_gcn, W_h, b_h, W_c, b_c):
    ids2 = node_indicator.astype(jnp.int32).reshape(N, 1)
    return _run(filtre, X, ids2, W_gcn, W_h,
                b_h.reshape(1, H), W_c, b_c.reshape(1, C))
